# Initial kernel scaffold; baseline (speedup 1.0000x reference)
import jax
import jax.numpy as jnp
from jax.experimental import pallas as pl


def kernel(v, e, W1n, b1n, W2n, b2n, W1e, b1e, W2e, b2e, edge_list, num_nodes):
    raise NotImplementedError("write your pallas kernel here")


# speedup vs baseline: 2.4562x; 2.4562x over previous
import dataclasses

import jax
import jax.numpy as jnp
from jax import lax
from jax.experimental import pallas as pl
from jax.experimental.pallas import tpu as pltpu
from jax.experimental.pallas import tpu_sc as plsc

V = 10000
E = 320000
D = 128
ED = 16
NC = 2
NS = 16
NW = NC * NS
EPW = E // NW
CHUNK = 80
NCH = EPW // CHUNK
V_PAD = 10240
VPS = V_PAD // NS

_SC_MESH = plsc.VectorSubcoreMesh(core_axis_name="c", subcore_axis_name="s")

_SC_PARAMS = pltpu.CompilerParams()
if "needs_layout_passes" in pltpu.CompilerParams.__dataclass_fields__:
    _SC_PARAMS = dataclasses.replace(_SC_PARAMS, needs_layout_passes=False)


def _gather_body(v_hbm, src_hbm, trm_hbm, vs_hbm, vt_hbm,
                 idx_s, idx_t, rows_s, rows_t, sem_s, sem_t):
    cid = lax.axis_index("c")
    sid = lax.axis_index("s")
    wid = sid * NC + cid
    base = wid * EPW

    @pl.loop(0, NCH)
    def _(ch):
        off = base + ch * CHUNK
        pltpu.sync_copy(src_hbm.at[pl.ds(off, CHUNK)], idx_s)
        pltpu.sync_copy(trm_hbm.at[pl.ds(off, CHUNK)], idx_t)
        cp_s = pltpu.async_copy(v_hbm.at[idx_s], rows_s, sem_s)
        cp_t = pltpu.async_copy(v_hbm.at[idx_t], rows_t, sem_t)
        cp_s.wait()
        cp_t.wait()
        pltpu.sync_copy(rows_s, vs_hbm.at[pl.ds(off, CHUNK)])
        pltpu.sync_copy(rows_t, vt_hbm.at[pl.ds(off, CHUNK)])


_sc_gather = pl.kernel(
    _gather_body,
    out_type=(jax.ShapeDtypeStruct((E, D), jnp.float32),
              jax.ShapeDtypeStruct((E, D), jnp.float32)),
    mesh=_SC_MESH,
    scratch_types=[
        pltpu.VMEM((CHUNK,), jnp.int32),
        pltpu.VMEM((CHUNK,), jnp.int32),
        pltpu.VMEM((CHUNK, D), jnp.float32),
        pltpu.VMEM((CHUNK, D), jnp.float32),
        pltpu.SemaphoreType.DMA,
        pltpu.SemaphoreType.DMA,
    ],
)


def _mlp_body(vs_ref, vt_ref, e_ref, w1s_ref, w1t_ref, w1e_ref, b1_ref,
              w2n_ref, w2e_ref, b2n_ref, b2e_ref, y_ref, oute_ref):
    h = (jnp.dot(vs_ref[...], w1s_ref[...], preferred_element_type=jnp.float32)
         + jnp.dot(vt_ref[...], w1t_ref[...], preferred_element_type=jnp.float32)
         + jnp.dot(e_ref[...], w1e_ref[...], preferred_element_type=jnp.float32)
         + b1_ref[...])
    h = jnp.maximum(h, 0.0)
    y_ref[...] = jnp.dot(h[:, :D], w2n_ref[...],
                         preferred_element_type=jnp.float32) + b2n_ref[...]
    ue = jnp.dot(h[:, D:], w2e_ref[...],
                 preferred_element_type=jnp.float32) + b2e_ref[...]
    oute_ref[...] = jnp.maximum(ue, 0.0)


def _run_mlp(vs, vt, e, w1s, w1t, w1e, b1, w2n, w2e, b2n, b2e, block_e=2000):
    grid = (E // block_e,)
    full = lambda i: (0, 0)
    return pl.pallas_call(
        _mlp_body,
        grid=grid,
        in_specs=[
            pl.BlockSpec((block_e, D), lambda i: (i, 0)),
            pl.BlockSpec((block_e, D), lambda i: (i, 0)),
            pl.BlockSpec((block_e, ED), lambda i: (i, 0)),
            pl.BlockSpec((D, 2 * D), full),
            pl.BlockSpec((D, 2 * D), full),
            pl.BlockSpec((ED, 2 * D), full),
            pl.BlockSpec((1, 2 * D), full),
            pl.BlockSpec((D, D), full),
            pl.BlockSpec((D, D), full),
            pl.BlockSpec((1, D), full),
            pl.BlockSpec((1, D), full),
        ],
        out_specs=[
            pl.BlockSpec((block_e, D), lambda i: (i, 0)),
            pl.BlockSpec((block_e, D), lambda i: (i, 0)),
        ],
        out_shape=[
            jax.ShapeDtypeStruct((E, D), jnp.float32),
            jax.ShapeDtypeStruct((E, D), jnp.float32),
        ],
    )(vs, vt, e, w1s, w1t, w1e, b1, w2n, w2e, b2n, b2e)


_ONES16 = None


def _scatter_body(y_hbm, trm_hbm, zrows_hbm, zdeg_hbm, out_hbm, deg_hbm,
                  acc_sh, idx_v, rows_v, deg_v):
    cid = lax.axis_index("c")
    sid = lax.axis_index("s")
    wid = sid * NC + cid
    base = wid * EPW
    pltpu.sync_copy(zrows_hbm, acc_sh.at[pl.ds(sid * VPS, VPS)])
    pltpu.sync_copy(zdeg_hbm, deg_v)
    plsc.subcore_barrier()

    ones = jnp.ones((16,), jnp.float32)

    @pl.loop(0, NCH)
    def _(ch):
        off = base + ch * CHUNK
        pltpu.sync_copy(trm_hbm.at[pl.ds(off, CHUNK)], idx_v)
        pltpu.sync_copy(y_hbm.at[pl.ds(off, CHUNK)], rows_v)
        pltpu.sync_copy(rows_v, acc_sh.at[idx_v], add=True)
        for j in range(CHUNK // 16):
            iv = idx_v[pl.ds(j * 16, 16)]
            plsc.addupdate_scatter(deg_v, [iv], ones)

    plsc.subcore_barrier()
    pltpu.sync_copy(acc_sh.at[pl.ds(sid * VPS, VPS)],
                    out_hbm.at[cid, pl.ds(sid * VPS, VPS)])
    pltpu.sync_copy(deg_v, deg_hbm.at[pl.ds(wid * V_PAD, V_PAD)])


_sc_scatter = pl.kernel(
    _scatter_body,
    out_type=(jax.ShapeDtypeStruct((NC, V_PAD, D), jnp.float32),
              jax.ShapeDtypeStruct((NW * V_PAD,), jnp.float32)),
    mesh=_SC_MESH,
    scratch_types=[
        pltpu.VMEM_SHARED((V_PAD, D), jnp.float32),
        pltpu.VMEM((CHUNK,), jnp.int32),
        pltpu.VMEM((CHUNK, D), jnp.float32),
        pltpu.VMEM((V_PAD,), jnp.float32),
    ],
    compiler_params=_SC_PARAMS,
)


def _final_body(p_ref, degp_ref, out_ref):
    s = p_ref[0] + p_ref[1]
    ones_col = jnp.ones((NW, 1), jnp.float32)
    dcol = lax.dot_general(degp_ref[...], ones_col, (((0,), (0,)), ((), ())),
                           preferred_element_type=jnp.float32)
    deg = jnp.maximum(dcol, 1.0)
    out_ref[...] = jnp.maximum(s / deg, 0.0)


def _run_final(partials, degp, block_v=1280):
    return pl.pallas_call(
        _final_body,
        grid=(V_PAD // block_v,),
        in_specs=[
            pl.BlockSpec((NC, block_v, D), lambda i: (0, i, 0)),
            pl.BlockSpec((NW, block_v), lambda i: (0, i)),
        ],
        out_specs=pl.BlockSpec((block_v, D), lambda i: (i, 0)),
        out_shape=jax.ShapeDtypeStruct((V_PAD, D), jnp.float32),
    )(partials, degp)


def kernel(v, e, W1n, b1n, W2n, b2n, W1e, b1e, W2e, b2e, edge_list, num_nodes):
    v2 = v.reshape(V, D)
    e2 = e.reshape(E, ED)
    sour = edge_list[0].astype(jnp.int32)
    term = edge_list[1].astype(jnp.int32)

    w1s = jnp.concatenate([W1n[:D], W1e[:D]], axis=1)
    w1t = jnp.concatenate([W1n[D:2 * D], W1e[D:2 * D]], axis=1)
    w1e = jnp.concatenate([W1n[2 * D:], W1e[2 * D:]], axis=1)
    b1 = jnp.concatenate([b1n, b1e]).reshape(1, 2 * D)
    b2n2 = b2n.reshape(1, D)
    b2e2 = b2e.reshape(1, D)

    vs, vt = _sc_gather(v2, sour, term)
    y, out_e = _run_mlp(vs, vt, e2, w1s, w1t, w1e, b1, W2n, W2e, b2n2, b2e2)
    zrows = jnp.zeros((VPS, D), jnp.float32)
    zdeg = jnp.zeros((V_PAD,), jnp.float32)
    partials, degp = _sc_scatter(y, term, zrows, zdeg)
    out_v = _run_final(partials, degp.reshape(NW, V_PAD))[:V]
    return out_v.reshape(1, V, D), out_e.reshape(1, E, D)


# speedup vs baseline: 2.8068x; 1.1427x over previous
import dataclasses

import jax
import jax.numpy as jnp
from jax import lax
from jax.experimental import pallas as pl
from jax.experimental.pallas import tpu as pltpu
from jax.experimental.pallas import tpu_sc as plsc

V = 10000
E = 320000
D = 128
ED = 16
NC = 2
NS = 16
NW = NC * NS
K = 5
EK = E // K
EPWK = EK // NW
EPW = E // NW
CHUNK = 80
NCHK = EPWK // CHUNK
NCH = EPW // CHUNK
V_PAD = 10240
VPS = V_PAD // NS

_SC_MESH = plsc.VectorSubcoreMesh(core_axis_name="c", subcore_axis_name="s")

_SC_PARAMS = pltpu.CompilerParams()
if "needs_layout_passes" in pltpu.CompilerParams.__dataclass_fields__:
    _SC_PARAMS = dataclasses.replace(_SC_PARAMS, needs_layout_passes=False)


def _gather_body(v_hbm, src_hbm, trm_hbm, vs_hbm, vt_hbm,
                 idx_s, idx_t, rows_s, rows_t, sem_s, sem_t):
    cid = lax.axis_index("c")
    sid = lax.axis_index("s")
    wid = sid * NC + cid
    base = wid * EPWK

    @pl.loop(0, NCHK)
    def _(ch):
        off = base + ch * CHUNK
        pltpu.sync_copy(src_hbm.at[pl.ds(off, CHUNK)], idx_s)
        pltpu.sync_copy(trm_hbm.at[pl.ds(off, CHUNK)], idx_t)
        cp_s = pltpu.async_copy(v_hbm.at[idx_s], rows_s, sem_s)
        cp_t = pltpu.async_copy(v_hbm.at[idx_t], rows_t, sem_t)
        cp_s.wait()
        cp_t.wait()
        pltpu.sync_copy(rows_s, vs_hbm.at[pl.ds(off, CHUNK)])
        pltpu.sync_copy(rows_t, vt_hbm.at[pl.ds(off, CHUNK)])


_sc_gather = pl.kernel(
    _gather_body,
    out_type=(jax.ShapeDtypeStruct((EK, D), jnp.float32),
              jax.ShapeDtypeStruct((EK, D), jnp.float32)),
    mesh=_SC_MESH,
    scratch_types=[
        pltpu.VMEM((CHUNK,), jnp.int32),
        pltpu.VMEM((CHUNK,), jnp.int32),
        pltpu.VMEM((CHUNK, D), jnp.float32),
        pltpu.VMEM((CHUNK, D), jnp.float32),
        pltpu.SemaphoreType.DMA,
        pltpu.SemaphoreType.DMA,
    ],
)


def _mlp_body(vs_ref, vt_ref, e_ref, w1s_ref, w1t_ref, w1e_ref, b1_ref,
              w2n_ref, w2e_ref, b2n_ref, b2e_ref, y_ref, oute_ref):
    h = (jnp.dot(vs_ref[...], w1s_ref[...], preferred_element_type=jnp.float32)
         + jnp.dot(vt_ref[...], w1t_ref[...], preferred_element_type=jnp.float32)
         + jnp.dot(e_ref[...], w1e_ref[...], preferred_element_type=jnp.float32)
         + b1_ref[...])
    h = jnp.maximum(h, 0.0)
    y_ref[...] = jnp.dot(h[:, :D], w2n_ref[...],
                         preferred_element_type=jnp.float32) + b2n_ref[...]
    ue = jnp.dot(h[:, D:], w2e_ref[...],
                 preferred_element_type=jnp.float32) + b2e_ref[...]
    oute_ref[...] = jnp.maximum(ue, 0.0)


def _run_mlp_chunk(k, vs, vt, e, w1s, w1t, w1e, b1, w2n, w2e, b2n, b2e,
                   y_in=None, oute_in=None, block_e=2000):
    grid = (EK // block_e,)
    full = lambda i: (0, 0)
    blk0 = k * (EK // block_e)
    in_specs = [
        pl.BlockSpec((block_e, D), lambda i: (i, 0)),
        pl.BlockSpec((block_e, D), lambda i: (i, 0)),
        pl.BlockSpec((block_e, ED), lambda i: (i + blk0, 0)),
        pl.BlockSpec((D, 2 * D), full),
        pl.BlockSpec((D, 2 * D), full),
        pl.BlockSpec((ED, 2 * D), full),
        pl.BlockSpec((1, 2 * D), full),
        pl.BlockSpec((D, D), full),
        pl.BlockSpec((D, D), full),
        pl.BlockSpec((1, D), full),
        pl.BlockSpec((1, D), full),
    ]
    args = [vs, vt, e, w1s, w1t, w1e, b1, w2n, w2e, b2n, b2e]
    kwargs = {}
    if y_in is not None:
        in_specs += [pl.BlockSpec(memory_space=pl.ANY),
                     pl.BlockSpec(memory_space=pl.ANY)]
        args += [y_in, oute_in]
        kwargs["input_output_aliases"] = {11: 0, 12: 1}

    def body(*refs):
        _mlp_body(*refs[:11], *refs[-2:])

    return pl.pallas_call(
        body,
        grid=grid,
        in_specs=in_specs,
        out_specs=[
            pl.BlockSpec((block_e, D), lambda i: (i + blk0, 0)),
            pl.BlockSpec((block_e, D), lambda i: (i + blk0, 0)),
        ],
        out_shape=[
            jax.ShapeDtypeStruct((E, D), jnp.float32),
            jax.ShapeDtypeStruct((E, D), jnp.float32),
        ],
        **kwargs,
    )(*args)


_ONES16 = None


def _scatter_body(y_hbm, trm_hbm, zrows_hbm, zdeg_hbm, out_hbm, deg_hbm,
                  acc_sh, idx_v, rows_v, deg_v):
    cid = lax.axis_index("c")
    sid = lax.axis_index("s")
    wid = sid * NC + cid
    base = wid * EPW
    pltpu.sync_copy(zrows_hbm, acc_sh.at[pl.ds(sid * VPS, VPS)])
    pltpu.sync_copy(zdeg_hbm, deg_v)
    plsc.subcore_barrier()

    ones = jnp.ones((16,), jnp.float32)

    @pl.loop(0, NCH)
    def _(ch):
        off = base + ch * CHUNK
        pltpu.sync_copy(trm_hbm.at[pl.ds(off, CHUNK)], idx_v)
        pltpu.sync_copy(y_hbm.at[pl.ds(off, CHUNK)], rows_v)
        pltpu.sync_copy(rows_v, acc_sh.at[idx_v], add=True)
        for j in range(CHUNK // 16):
            iv = idx_v[pl.ds(j * 16, 16)]
            plsc.addupdate_scatter(deg_v, [iv], ones)

    plsc.subcore_barrier()
    pltpu.sync_copy(acc_sh.at[pl.ds(sid * VPS, VPS)],
                    out_hbm.at[cid, pl.ds(sid * VPS, VPS)])
    pltpu.sync_copy(deg_v, deg_hbm.at[pl.ds(wid * V_PAD, V_PAD)])


_sc_scatter = pl.kernel(
    _scatter_body,
    out_type=(jax.ShapeDtypeStruct((NC, V_PAD, D), jnp.float32),
              jax.ShapeDtypeStruct((NW * V_PAD,), jnp.float32)),
    mesh=_SC_MESH,
    scratch_types=[
        pltpu.VMEM_SHARED((V_PAD, D), jnp.float32),
        pltpu.VMEM((CHUNK,), jnp.int32),
        pltpu.VMEM((CHUNK, D), jnp.float32),
        pltpu.VMEM((V_PAD,), jnp.float32),
    ],
    compiler_params=_SC_PARAMS,
)


def _final_body(p_ref, degp_ref, out_ref):
    s = p_ref[0] + p_ref[1]
    ones_col = jnp.ones((NW, 1), jnp.float32)
    dcol = lax.dot_general(degp_ref[...], ones_col, (((0,), (0,)), ((), ())),
                           preferred_element_type=jnp.float32)
    deg = jnp.maximum(dcol, 1.0)
    out_ref[...] = jnp.maximum(s / deg, 0.0)


def _run_final(partials, degp, block_v=1280):
    return pl.pallas_call(
        _final_body,
        grid=(V_PAD // block_v,),
        in_specs=[
            pl.BlockSpec((NC, block_v, D), lambda i: (0, i, 0)),
            pl.BlockSpec((NW, block_v), lambda i: (0, i)),
        ],
        out_specs=pl.BlockSpec((block_v, D), lambda i: (i, 0)),
        out_shape=jax.ShapeDtypeStruct((V_PAD, D), jnp.float32),
    )(partials, degp)


def kernel(v, e, W1n, b1n, W2n, b2n, W1e, b1e, W2e, b2e, edge_list, num_nodes):
    v2 = v.reshape(V, D)
    e2 = e.reshape(E, ED)
    sour = edge_list[0].astype(jnp.int32)
    term = edge_list[1].astype(jnp.int32)

    w1s = jnp.concatenate([W1n[:D], W1e[:D]], axis=1)
    w1t = jnp.concatenate([W1n[D:2 * D], W1e[D:2 * D]], axis=1)
    w1e = jnp.concatenate([W1n[2 * D:], W1e[2 * D:]], axis=1)
    b1 = jnp.concatenate([b1n, b1e]).reshape(1, 2 * D)
    b2n2 = b2n.reshape(1, D)
    b2e2 = b2e.reshape(1, D)

    y = out_e = None
    for k in range(K):
        sour_k = lax.slice(sour, (k * EK,), ((k + 1) * EK,))
        term_k = lax.slice(term, (k * EK,), ((k + 1) * EK,))
        vs_k, vt_k = _sc_gather(v2, sour_k, term_k)
        y, out_e = _run_mlp_chunk(k, vs_k, vt_k, e2, w1s, w1t, w1e, b1,
                                  W2n, W2e, b2n2, b2e2, y, out_e)
    zrows = jnp.zeros((VPS, D), jnp.float32)
    zdeg = jnp.zeros((V_PAD,), jnp.float32)
    partials, degp = _sc_scatter(y, term, zrows, zdeg)
    out_v = _run_final(partials, degp.reshape(NW, V_PAD))[:V]
    return out_v.reshape(1, V, D), out_e.reshape(1, E, D)


# speedup vs baseline: 3.5261x; 1.2563x over previous
import dataclasses

import jax
import jax.numpy as jnp
from jax import lax
from jax.experimental import pallas as pl
from jax.experimental.pallas import tpu as pltpu
from jax.experimental.pallas import tpu_sc as plsc

V = 10000
E = 320000
D = 128
ED = 16
NC = 2
NS = 16
NW = NC * NS
K = 5
EK = E // K
EPWK = EK // NW
EPW = E // NW
CHUNK = 80
NCHK = EPWK // CHUNK
NCH = EPW // CHUNK
V_PAD = 10240
VPS = V_PAD // NS

_SC_MESH = plsc.VectorSubcoreMesh(core_axis_name="c", subcore_axis_name="s")

_SC_PARAMS = pltpu.CompilerParams()
if "needs_layout_passes" in pltpu.CompilerParams.__dataclass_fields__:
    _SC_PARAMS = dataclasses.replace(_SC_PARAMS, needs_layout_passes=False)


def _gather_body(v_hbm, src_hbm, trm_hbm, vs_hbm, vt_hbm,
                 idx_s0, idx_t0, idx_s1, idx_t1,
                 rows_s0, rows_t0, rows_s1, rows_t1,
                 sem_g0, sem_g1, sem_w0, sem_w1):
    cid = lax.axis_index("c")
    sid = lax.axis_index("s")
    wid = sid * NC + cid
    base = wid * EPWK
    idx_b = ((idx_s0, idx_t0), (idx_s1, idx_t1))
    row_b = ((rows_s0, rows_t0), (rows_s1, rows_t1))
    sem_g = (sem_g0, sem_g1)
    sem_w = (sem_w0, sem_w1)

    def load_idx(ch, b):
        off = base + ch * CHUNK
        pltpu.sync_copy(src_hbm.at[pl.ds(off, CHUNK)], idx_b[b][0])
        pltpu.sync_copy(trm_hbm.at[pl.ds(off, CHUNK)], idx_b[b][1])

    def gstart(b):
        pltpu.make_async_copy(v_hbm.at[idx_b[b][0]], row_b[b][0], sem_g[b]).start()
        pltpu.make_async_copy(v_hbm.at[idx_b[b][1]], row_b[b][1], sem_g[b]).start()

    def gwait(b):
        pltpu.make_async_copy(v_hbm.at[idx_b[b][0]], row_b[b][0], sem_g[b]).wait()
        pltpu.make_async_copy(v_hbm.at[idx_b[b][1]], row_b[b][1], sem_g[b]).wait()

    def wstart(ch, b):
        off = base + ch * CHUNK
        pltpu.make_async_copy(row_b[b][0], vs_hbm.at[pl.ds(off, CHUNK)], sem_w[b]).start()
        pltpu.make_async_copy(row_b[b][1], vt_hbm.at[pl.ds(off, CHUNK)], sem_w[b]).start()

    def wwait(b):
        pltpu.make_async_copy(row_b[b][0], vs_hbm.at[pl.ds(base, CHUNK)], sem_w[b]).wait()
        pltpu.make_async_copy(row_b[b][1], vt_hbm.at[pl.ds(base, CHUNK)], sem_w[b]).wait()

    load_idx(0, 0)
    gstart(0)

    @pl.loop(0, NCHK - 1, step=2)
    def _(ch):
        load_idx(ch + 1, 1)
        gwait(0)

        @pl.when(ch > 0)
        def _():
            wwait(1)

        gstart(1)
        wstart(ch, 0)
        load_idx(ch + 2, 0)
        gwait(1)
        wwait(0)
        gstart(0)
        wstart(ch + 1, 1)

    gwait(0)
    wwait(1)
    wstart(NCHK - 1, 0)
    wwait(0)


_sc_gather = pl.kernel(
    _gather_body,
    out_type=(jax.ShapeDtypeStruct((EK, D), jnp.float32),
              jax.ShapeDtypeStruct((EK, D), jnp.float32)),
    mesh=_SC_MESH,
    scratch_types=[
        pltpu.VMEM((CHUNK,), jnp.int32),
        pltpu.VMEM((CHUNK,), jnp.int32),
        pltpu.VMEM((CHUNK,), jnp.int32),
        pltpu.VMEM((CHUNK,), jnp.int32),
        pltpu.VMEM((CHUNK, D), jnp.float32),
        pltpu.VMEM((CHUNK, D), jnp.float32),
        pltpu.VMEM((CHUNK, D), jnp.float32),
        pltpu.VMEM((CHUNK, D), jnp.float32),
        pltpu.SemaphoreType.DMA,
        pltpu.SemaphoreType.DMA,
        pltpu.SemaphoreType.DMA,
        pltpu.SemaphoreType.DMA,
    ],
)


def _mlp_body(vs_ref, vt_ref, e_ref, w1s_ref, w1t_ref, w1e_ref, b1_ref,
              w2n_ref, w2e_ref, b2n_ref, b2e_ref, y_ref, oute_ref):
    h = (jnp.dot(vs_ref[...], w1s_ref[...], preferred_element_type=jnp.float32)
         + jnp.dot(vt_ref[...], w1t_ref[...], preferred_element_type=jnp.float32)
         + jnp.dot(e_ref[...], w1e_ref[...], preferred_element_type=jnp.float32)
         + b1_ref[...])
    h = jnp.maximum(h, 0.0)
    y_ref[...] = jnp.dot(h[:, :D], w2n_ref[...],
                         preferred_element_type=jnp.float32) + b2n_ref[...]
    ue = jnp.dot(h[:, D:], w2e_ref[...],
                 preferred_element_type=jnp.float32) + b2e_ref[...]
    oute_ref[...] = jnp.maximum(ue, 0.0)


def _run_mlp_chunk(k, vs, vt, e, w1s, w1t, w1e, b1, w2n, w2e, b2n, b2e,
                   y_in=None, oute_in=None, block_e=2000):
    grid = (EK // block_e,)
    full = lambda i: (0, 0)
    blk0 = k * (EK // block_e)
    in_specs = [
        pl.BlockSpec((block_e, D), lambda i: (i, 0)),
        pl.BlockSpec((block_e, D), lambda i: (i, 0)),
        pl.BlockSpec((block_e, ED), lambda i: (i + blk0, 0)),
        pl.BlockSpec((D, 2 * D), full),
        pl.BlockSpec((D, 2 * D), full),
        pl.BlockSpec((ED, 2 * D), full),
        pl.BlockSpec((1, 2 * D), full),
        pl.BlockSpec((D, D), full),
        pl.BlockSpec((D, D), full),
        pl.BlockSpec((1, D), full),
        pl.BlockSpec((1, D), full),
    ]
    args = [vs, vt, e, w1s, w1t, w1e, b1, w2n, w2e, b2n, b2e]
    kwargs = {}
    if y_in is not None:
        in_specs += [pl.BlockSpec(memory_space=pl.ANY),
                     pl.BlockSpec(memory_space=pl.ANY)]
        args += [y_in, oute_in]
        kwargs["input_output_aliases"] = {11: 0, 12: 1}

    def body(*refs):
        _mlp_body(*refs[:11], *refs[-2:])

    return pl.pallas_call(
        body,
        grid=grid,
        in_specs=in_specs,
        out_specs=[
            pl.BlockSpec((block_e, D), lambda i: (i + blk0, 0)),
            pl.BlockSpec((block_e, D), lambda i: (i + blk0, 0)),
        ],
        out_shape=[
            jax.ShapeDtypeStruct((E, D), jnp.float32),
            jax.ShapeDtypeStruct((E, D), jnp.float32),
        ],
        **kwargs,
    )(*args)


_ONES16 = None


def _scatter_body(y_hbm, trm_hbm, zrows_hbm, zdeg_hbm, out_hbm, deg_hbm,
                  acc_sh, idx_v0, idx_v1, rows_v0, rows_v1, deg_v,
                  sem_l0, sem_l1, sem_a0, sem_a1):
    cid = lax.axis_index("c")
    sid = lax.axis_index("s")
    wid = sid * NC + cid
    base = wid * EPW
    idx_b = (idx_v0, idx_v1)
    row_b = (rows_v0, rows_v1)
    sem_l = (sem_l0, sem_l1)
    sem_a = (sem_a0, sem_a1)
    ones = jnp.ones((16,), jnp.float32)

    def lstart(ch, b):
        off = base + ch * CHUNK
        pltpu.make_async_copy(trm_hbm.at[pl.ds(off, CHUNK)], idx_b[b], sem_l[b]).start()
        pltpu.make_async_copy(y_hbm.at[pl.ds(off, CHUNK)], row_b[b], sem_l[b]).start()

    def lwait(b):
        pltpu.make_async_copy(trm_hbm.at[pl.ds(base, CHUNK)], idx_b[b], sem_l[b]).wait()
        pltpu.make_async_copy(y_hbm.at[pl.ds(base, CHUNK)], row_b[b], sem_l[b]).wait()

    def astart(b):
        pltpu.async_copy(row_b[b], acc_sh.at[idx_b[b]], sem_a[b], add=True)

    def await_(b):
        pltpu.make_async_copy(row_b[b], acc_sh.at[idx_b[b]], sem_a[b]).wait()

    def deg_update(b):
        for j in range(CHUNK // 16):
            iv = idx_b[b][pl.ds(j * 16, 16)]
            plsc.addupdate_scatter(deg_v, [iv], ones)

    pltpu.sync_copy(zrows_hbm, acc_sh.at[pl.ds(sid * VPS, VPS)])
    pltpu.sync_copy(zdeg_hbm, deg_v)
    plsc.subcore_barrier()

    lstart(0, 0)

    @pl.loop(0, NCH - 1, step=2)
    def _(ch):
        lwait(0)

        @pl.when(ch > 0)
        def _():
            await_(1)

        lstart(ch + 1, 1)
        astart(0)
        deg_update(0)
        lwait(1)
        await_(0)
        lstart(ch + 2, 0)
        astart(1)
        deg_update(1)

    lwait(0)
    await_(1)
    astart(0)
    deg_update(0)
    await_(0)

    plsc.subcore_barrier()
    pltpu.sync_copy(acc_sh.at[pl.ds(sid * VPS, VPS)],
                    out_hbm.at[cid, pl.ds(sid * VPS, VPS)])
    pltpu.sync_copy(deg_v, deg_hbm.at[pl.ds(wid * V_PAD, V_PAD)])


_sc_scatter = pl.kernel(
    _scatter_body,
    out_type=(jax.ShapeDtypeStruct((NC, V_PAD, D), jnp.float32),
              jax.ShapeDtypeStruct((NW * V_PAD,), jnp.float32)),
    mesh=_SC_MESH,
    scratch_types=[
        pltpu.VMEM_SHARED((V_PAD, D), jnp.float32),
        pltpu.VMEM((CHUNK,), jnp.int32),
        pltpu.VMEM((CHUNK,), jnp.int32),
        pltpu.VMEM((CHUNK, D), jnp.float32),
        pltpu.VMEM((CHUNK, D), jnp.float32),
        pltpu.VMEM((V_PAD,), jnp.float32),
        pltpu.SemaphoreType.DMA,
        pltpu.SemaphoreType.DMA,
        pltpu.SemaphoreType.DMA,
        pltpu.SemaphoreType.DMA,
    ],
    compiler_params=_SC_PARAMS,
)


def _final_body(p_ref, degp_ref, out_ref):
    s = p_ref[0] + p_ref[1]
    ones_col = jnp.ones((NW, 1), jnp.float32)
    dcol = lax.dot_general(degp_ref[...], ones_col, (((0,), (0,)), ((), ())),
                           preferred_element_type=jnp.float32)
    deg = jnp.maximum(dcol, 1.0)
    out_ref[...] = jnp.maximum(s / deg, 0.0)


def _run_final(partials, degp, block_v=1280):
    return pl.pallas_call(
        _final_body,
        grid=(V_PAD // block_v,),
        in_specs=[
            pl.BlockSpec((NC, block_v, D), lambda i: (0, i, 0)),
            pl.BlockSpec((NW, block_v), lambda i: (0, i)),
        ],
        out_specs=pl.BlockSpec((block_v, D), lambda i: (i, 0)),
        out_shape=jax.ShapeDtypeStruct((V_PAD, D), jnp.float32),
    )(partials, degp)


def kernel(v, e, W1n, b1n, W2n, b2n, W1e, b1e, W2e, b2e, edge_list, num_nodes):
    v2 = v.reshape(V, D)
    e2 = e.reshape(E, ED)
    sour = edge_list[0].astype(jnp.int32)
    term = edge_list[1].astype(jnp.int32)

    w1s = jnp.concatenate([W1n[:D], W1e[:D]], axis=1)
    w1t = jnp.concatenate([W1n[D:2 * D], W1e[D:2 * D]], axis=1)
    w1e = jnp.concatenate([W1n[2 * D:], W1e[2 * D:]], axis=1)
    b1 = jnp.concatenate([b1n, b1e]).reshape(1, 2 * D)
    b2n2 = b2n.reshape(1, D)
    b2e2 = b2e.reshape(1, D)

    y = out_e = None
    for k in range(K):
        sour_k = lax.slice(sour, (k * EK,), ((k + 1) * EK,))
        term_k = lax.slice(term, (k * EK,), ((k + 1) * EK,))
        vs_k, vt_k = _sc_gather(v2, sour_k, term_k)
        y, out_e = _run_mlp_chunk(k, vs_k, vt_k, e2, w1s, w1t, w1e, b1,
                                  W2n, W2e, b2n2, b2e2, y, out_e)
    zrows = jnp.zeros((VPS, D), jnp.float32)
    zdeg = jnp.zeros((V_PAD,), jnp.float32)
    partials, degp = _sc_scatter(y, term, zrows, zdeg)
    out_v = _run_final(partials, degp.reshape(NW, V_PAD))[:V]
    return out_v.reshape(1, V, D), out_e.reshape(1, E, D)


# speedup vs baseline: 3.7454x; 1.0622x over previous
import dataclasses

import jax
import jax.numpy as jnp
from jax import lax
from jax.experimental import pallas as pl
from jax.experimental.pallas import tpu as pltpu
from jax.experimental.pallas import tpu_sc as plsc

V = 10000
E = 320000
D = 128
ED = 16
NC = 2
NS = 16
NW = NC * NS
K = 5
EK = E // K
EPWK = EK // NW
EPW = E // NW
CHUNK = 80
NCHK = EPWK // CHUNK
NCH = EPW // CHUNK
V_PAD = 10240
VPS = V_PAD // NS

_SC_MESH = plsc.VectorSubcoreMesh(core_axis_name="c", subcore_axis_name="s")

_SC_PARAMS = pltpu.CompilerParams()
if "needs_layout_passes" in pltpu.CompilerParams.__dataclass_fields__:
    _SC_PARAMS = dataclasses.replace(_SC_PARAMS, needs_layout_passes=False)


def _gather_body(v_hbm, src_hbm, trm_hbm, vs_hbm, vt_hbm,
                 v_sh, idx_s0, idx_t0, idx_s1, idx_t1,
                 rows_s0, rows_t0, rows_s1, rows_t1,
                 sem_g0, sem_g1, sem_w0, sem_w1):
    cid = lax.axis_index("c")
    sid = lax.axis_index("s")
    wid = sid * NC + cid
    base = wid * EPWK
    pltpu.sync_copy(v_hbm.at[pl.ds(sid * VPS, VPS)],
                    v_sh.at[pl.ds(sid * VPS, VPS)])
    plsc.subcore_barrier()
    idx_b = ((idx_s0, idx_t0), (idx_s1, idx_t1))
    row_b = ((rows_s0, rows_t0), (rows_s1, rows_t1))
    sem_g = (sem_g0, sem_g1)
    sem_w = (sem_w0, sem_w1)

    def load_idx(ch, b):
        off = base + ch * CHUNK
        pltpu.sync_copy(src_hbm.at[pl.ds(off, CHUNK)], idx_b[b][0])
        pltpu.sync_copy(trm_hbm.at[pl.ds(off, CHUNK)], idx_b[b][1])

    def gstart(b):
        pltpu.make_async_copy(v_sh.at[idx_b[b][0]], row_b[b][0], sem_g[b]).start()
        pltpu.make_async_copy(v_sh.at[idx_b[b][1]], row_b[b][1], sem_g[b]).start()

    def gwait(b):
        pltpu.make_async_copy(v_sh.at[idx_b[b][0]], row_b[b][0], sem_g[b]).wait()
        pltpu.make_async_copy(v_sh.at[idx_b[b][1]], row_b[b][1], sem_g[b]).wait()

    def wstart(ch, b):
        off = base + ch * CHUNK
        pltpu.make_async_copy(row_b[b][0], vs_hbm.at[pl.ds(off, CHUNK)], sem_w[b]).start()
        pltpu.make_async_copy(row_b[b][1], vt_hbm.at[pl.ds(off, CHUNK)], sem_w[b]).start()

    def wwait(b):
        pltpu.make_async_copy(row_b[b][0], vs_hbm.at[pl.ds(base, CHUNK)], sem_w[b]).wait()
        pltpu.make_async_copy(row_b[b][1], vt_hbm.at[pl.ds(base, CHUNK)], sem_w[b]).wait()

    load_idx(0, 0)
    gstart(0)

    @pl.loop(0, NCHK - 1, step=2)
    def _(ch):
        load_idx(ch + 1, 1)
        gwait(0)

        @pl.when(ch > 0)
        def _():
            wwait(1)

        gstart(1)
        wstart(ch, 0)
        load_idx(ch + 2, 0)
        gwait(1)
        wwait(0)
        gstart(0)
        wstart(ch + 1, 1)

    gwait(0)
    wwait(1)
    wstart(NCHK - 1, 0)
    wwait(0)


_sc_gather = pl.kernel(
    _gather_body,
    out_type=(jax.ShapeDtypeStruct((EK, D), jnp.float32),
              jax.ShapeDtypeStruct((EK, D), jnp.float32)),
    mesh=_SC_MESH,
    scratch_types=[
        pltpu.VMEM_SHARED((V_PAD, D), jnp.float32),
        pltpu.VMEM((CHUNK,), jnp.int32),
        pltpu.VMEM((CHUNK,), jnp.int32),
        pltpu.VMEM((CHUNK,), jnp.int32),
        pltpu.VMEM((CHUNK,), jnp.int32),
        pltpu.VMEM((CHUNK, D), jnp.float32),
        pltpu.VMEM((CHUNK, D), jnp.float32),
        pltpu.VMEM((CHUNK, D), jnp.float32),
        pltpu.VMEM((CHUNK, D), jnp.float32),
        pltpu.SemaphoreType.DMA,
        pltpu.SemaphoreType.DMA,
        pltpu.SemaphoreType.DMA,
        pltpu.SemaphoreType.DMA,
    ],
)


def _mlp_body(vs_ref, vt_ref, e_ref, w1s_ref, w1t_ref, w1e_ref, b1_ref,
              w2n_ref, w2e_ref, b2n_ref, b2e_ref, y_ref, oute_ref):
    vsb = vs_ref[...].astype(jnp.bfloat16)
    vtb = vt_ref[...].astype(jnp.bfloat16)
    eb = e_ref[...].astype(jnp.bfloat16)
    h = (jnp.dot(vsb, w1s_ref[...], preferred_element_type=jnp.float32)
         + jnp.dot(vtb, w1t_ref[...], preferred_element_type=jnp.float32)
         + jnp.dot(eb, w1e_ref[...], preferred_element_type=jnp.float32)
         + b1_ref[...])
    h = jnp.maximum(h, 0.0).astype(jnp.bfloat16)
    y_ref[...] = jnp.dot(h[:, :D], w2n_ref[...],
                         preferred_element_type=jnp.float32) + b2n_ref[...]
    ue = jnp.dot(h[:, D:], w2e_ref[...],
                 preferred_element_type=jnp.float32) + b2e_ref[...]
    oute_ref[...] = jnp.maximum(ue, 0.0)


def _run_mlp_chunk(k, vs, vt, e, w1s, w1t, w1e, b1, w2n, w2e, b2n, b2e,
                   y_in=None, oute_in=None, block_e=2000):
    grid = (EK // block_e,)
    full = lambda i: (0, 0)
    blk0 = k * (EK // block_e)
    in_specs = [
        pl.BlockSpec((block_e, D), lambda i: (i, 0)),
        pl.BlockSpec((block_e, D), lambda i: (i, 0)),
        pl.BlockSpec((block_e, ED), lambda i: (i + blk0, 0)),
        pl.BlockSpec((D, 2 * D), full),
        pl.BlockSpec((D, 2 * D), full),
        pl.BlockSpec((ED, 2 * D), full),
        pl.BlockSpec((1, 2 * D), full),
        pl.BlockSpec((D, D), full),
        pl.BlockSpec((D, D), full),
        pl.BlockSpec((1, D), full),
        pl.BlockSpec((1, D), full),
    ]
    args = [vs, vt, e, w1s, w1t, w1e, b1, w2n, w2e, b2n, b2e]
    kwargs = {}
    if y_in is not None:
        in_specs += [pl.BlockSpec(memory_space=pl.ANY),
                     pl.BlockSpec(memory_space=pl.ANY)]
        args += [y_in, oute_in]
        kwargs["input_output_aliases"] = {11: 0, 12: 1}

    def body(*refs):
        _mlp_body(*refs[:11], *refs[-2:])

    return pl.pallas_call(
        body,
        grid=grid,
        in_specs=in_specs,
        out_specs=[
            pl.BlockSpec((block_e, D), lambda i: (i + blk0, 0)),
            pl.BlockSpec((block_e, D), lambda i: (i + blk0, 0)),
        ],
        out_shape=[
            jax.ShapeDtypeStruct((E, D), jnp.float32),
            jax.ShapeDtypeStruct((E, D), jnp.float32),
        ],
        **kwargs,
    )(*args)


_ONES16 = None


def _scatter_body(y_hbm, trm_hbm, zrows_hbm, zdeg_hbm, out_hbm, deg_hbm,
                  acc_sh, idx_v0, idx_v1, rows_v0, rows_v1, deg_v,
                  sem_l0, sem_l1, sem_a0, sem_a1):
    cid = lax.axis_index("c")
    sid = lax.axis_index("s")
    wid = sid * NC + cid
    base = wid * EPW
    idx_b = (idx_v0, idx_v1)
    row_b = (rows_v0, rows_v1)
    sem_l = (sem_l0, sem_l1)
    sem_a = (sem_a0, sem_a1)
    ones = jnp.ones((16,), jnp.float32)

    def lstart(ch, b):
        off = base + ch * CHUNK
        pltpu.make_async_copy(trm_hbm.at[pl.ds(off, CHUNK)], idx_b[b], sem_l[b]).start()
        pltpu.make_async_copy(y_hbm.at[pl.ds(off, CHUNK)], row_b[b], sem_l[b]).start()

    def lwait(b):
        pltpu.make_async_copy(trm_hbm.at[pl.ds(base, CHUNK)], idx_b[b], sem_l[b]).wait()
        pltpu.make_async_copy(y_hbm.at[pl.ds(base, CHUNK)], row_b[b], sem_l[b]).wait()

    def astart(b):
        pltpu.async_copy(row_b[b], acc_sh.at[idx_b[b]], sem_a[b], add=True)

    def await_(b):
        pltpu.make_async_copy(row_b[b], acc_sh.at[idx_b[b]], sem_a[b]).wait()

    def deg_update(b):
        for j in range(CHUNK // 16):
            iv = idx_b[b][pl.ds(j * 16, 16)]
            plsc.addupdate_scatter(deg_v, [iv], ones)

    pltpu.sync_copy(zrows_hbm, acc_sh.at[pl.ds(sid * VPS, VPS)])
    pltpu.sync_copy(zdeg_hbm, deg_v)
    plsc.subcore_barrier()

    lstart(0, 0)

    @pl.loop(0, NCH - 1, step=2)
    def _(ch):
        lwait(0)

        @pl.when(ch > 0)
        def _():
            await_(1)

        lstart(ch + 1, 1)
        astart(0)
        deg_update(0)
        lwait(1)
        await_(0)
        lstart(ch + 2, 0)
        astart(1)
        deg_update(1)

    lwait(0)
    await_(1)
    astart(0)
    deg_update(0)
    await_(0)

    plsc.subcore_barrier()
    pltpu.sync_copy(acc_sh.at[pl.ds(sid * VPS, VPS)],
                    out_hbm.at[cid, pl.ds(sid * VPS, VPS)])
    pltpu.sync_copy(deg_v, deg_hbm.at[pl.ds(wid * V_PAD, V_PAD)])


_sc_scatter = pl.kernel(
    _scatter_body,
    out_type=(jax.ShapeDtypeStruct((NC, V_PAD, D), jnp.float32),
              jax.ShapeDtypeStruct((NW * V_PAD,), jnp.float32)),
    mesh=_SC_MESH,
    scratch_types=[
        pltpu.VMEM_SHARED((V_PAD, D), jnp.float32),
        pltpu.VMEM((CHUNK,), jnp.int32),
        pltpu.VMEM((CHUNK,), jnp.int32),
        pltpu.VMEM((CHUNK, D), jnp.float32),
        pltpu.VMEM((CHUNK, D), jnp.float32),
        pltpu.VMEM((V_PAD,), jnp.float32),
        pltpu.SemaphoreType.DMA,
        pltpu.SemaphoreType.DMA,
        pltpu.SemaphoreType.DMA,
        pltpu.SemaphoreType.DMA,
    ],
    compiler_params=_SC_PARAMS,
)


def _final_body(p_ref, degp_ref, out_ref):
    s = p_ref[0] + p_ref[1]
    ones_col = jnp.ones((NW, 1), jnp.float32)
    dcol = lax.dot_general(degp_ref[...], ones_col, (((0,), (0,)), ((), ())),
                           preferred_element_type=jnp.float32)
    deg = jnp.maximum(dcol, 1.0)
    out_ref[...] = jnp.maximum(s / deg, 0.0)


def _run_final(partials, degp, block_v=1280):
    return pl.pallas_call(
        _final_body,
        grid=(V_PAD // block_v,),
        in_specs=[
            pl.BlockSpec((NC, block_v, D), lambda i: (0, i, 0)),
            pl.BlockSpec((NW, block_v), lambda i: (0, i)),
        ],
        out_specs=pl.BlockSpec((block_v, D), lambda i: (i, 0)),
        out_shape=jax.ShapeDtypeStruct((V_PAD, D), jnp.float32),
    )(partials, degp)


def kernel(v, e, W1n, b1n, W2n, b2n, W1e, b1e, W2e, b2e, edge_list, num_nodes):
    v2 = v.reshape(V, D)
    e2 = e.reshape(E, ED)
    sour = edge_list[0].astype(jnp.int32)
    term = edge_list[1].astype(jnp.int32)

    w1s = jnp.concatenate([W1n[:D], W1e[:D]], axis=1).astype(jnp.bfloat16)
    w1t = jnp.concatenate([W1n[D:2 * D], W1e[D:2 * D]], axis=1).astype(jnp.bfloat16)
    w1e = jnp.concatenate([W1n[2 * D:], W1e[2 * D:]], axis=1).astype(jnp.bfloat16)
    b1 = jnp.concatenate([b1n, b1e]).reshape(1, 2 * D)
    b2n2 = b2n.reshape(1, D)
    b2e2 = b2e.reshape(1, D)
    W2nb = W2n.astype(jnp.bfloat16)
    W2eb = W2e.astype(jnp.bfloat16)

    vpad = jnp.pad(v2, ((0, V_PAD - V), (0, 0)))
    y = out_e = None
    for k in range(K):
        sour_k = lax.slice(sour, (k * EK,), ((k + 1) * EK,))
        term_k = lax.slice(term, (k * EK,), ((k + 1) * EK,))
        vs_k, vt_k = _sc_gather(vpad, sour_k, term_k)
        y, out_e = _run_mlp_chunk(k, vs_k, vt_k, e2, w1s, w1t, w1e, b1,
                                  W2nb, W2eb, b2n2, b2e2, y, out_e)
    zrows = jnp.zeros((VPS, D), jnp.float32)
    zdeg = jnp.zeros((V_PAD,), jnp.float32)
    partials, degp = _sc_scatter(y, term, zrows, zdeg)
    out_v = _run_final(partials, degp.reshape(NW, V_PAD))[:V]
    return out_v.reshape(1, V, D), out_e.reshape(1, E, D)


# speedup vs baseline: 4.1160x; 1.0989x over previous
import dataclasses

import jax
import jax.numpy as jnp
from jax import lax
from jax.experimental import pallas as pl
from jax.experimental.pallas import tpu as pltpu
from jax.experimental.pallas import tpu_sc as plsc

V = 10000
E = 320000
D = 128
ED = 16
NC = 2
NS = 16
NW = NC * NS
K = 5
EK = E // K
EPWK = EK // NW
EPW = E // NW
CHUNK = 80
NCHK = EPWK // CHUNK
NCH = EPW // CHUNK
V_PAD = 10240
VPS = V_PAD // NS

_SC_MESH = plsc.VectorSubcoreMesh(core_axis_name="c", subcore_axis_name="s")

_SC_PARAMS = pltpu.CompilerParams()
if "needs_layout_passes" in pltpu.CompilerParams.__dataclass_fields__:
    _SC_PARAMS = dataclasses.replace(_SC_PARAMS, needs_layout_passes=False)


def _pack_pair(x):
    xb = x.astype(jnp.bfloat16)
    lo = lax.bitcast_convert_type(xb[:, :D], jnp.uint16).astype(jnp.uint32)
    hi = lax.bitcast_convert_type(xb[:, D:], jnp.uint16).astype(jnp.uint32)
    return lax.bitcast_convert_type(lo | (hi << 16), jnp.int32)


def _pre_body(v_ref, ws_ref, wt_ref, b1_ref, pp_ref, qp_ref):
    xb = v_ref[...].astype(jnp.bfloat16)
    p = jnp.dot(xb, ws_ref[...], preferred_element_type=jnp.float32) + b1_ref[...]
    q = jnp.dot(xb, wt_ref[...], preferred_element_type=jnp.float32)
    pp_ref[...] = _pack_pair(p)
    qp_ref[...] = _pack_pair(q)


def _run_pre(vpad, ws, wt, b1, block_v=2048):
    full = lambda i: (0, 0)
    return pl.pallas_call(
        _pre_body,
        grid=(V_PAD // block_v,),
        in_specs=[
            pl.BlockSpec((block_v, D), lambda i: (i, 0)),
            pl.BlockSpec((D, 2 * D), full),
            pl.BlockSpec((D, 2 * D), full),
            pl.BlockSpec((1, 2 * D), full),
        ],
        out_specs=[
            pl.BlockSpec((block_v, D), lambda i: (i, 0)),
            pl.BlockSpec((block_v, D), lambda i: (i, 0)),
        ],
        out_shape=[
            jax.ShapeDtypeStruct((V_PAD, D), jnp.int32),
            jax.ShapeDtypeStruct((V_PAD, D), jnp.int32),
        ],
    )(vpad, ws, wt, b1)


EPWK2 = EK // NS
NCHK2 = EPWK2 // CHUNK


def _gather_body(p_hbm, q_hbm, src_hbm, trm_hbm, gs_hbm, gt_hbm,
                 tab_sh, idx0, idx1, rows0, rows1,
                 sem_g0, sem_g1, sem_w0, sem_w1):
    cid = lax.axis_index("c")
    sid = lax.axis_index("s")
    base = sid * EPWK2
    idx_b = (idx0, idx1)
    row_b = (rows0, rows1)
    sem_g = (sem_g0, sem_g1)
    sem_w = (sem_w0, sem_w1)

    @pl.when(cid == 0)
    def _():
        pltpu.sync_copy(p_hbm.at[pl.ds(sid * VPS, VPS)],
                        tab_sh.at[pl.ds(sid * VPS, VPS)])

    @pl.when(cid == 1)
    def _():
        pltpu.sync_copy(q_hbm.at[pl.ds(sid * VPS, VPS)],
                        tab_sh.at[pl.ds(sid * VPS, VPS)])

    plsc.subcore_barrier()

    def ring(i_hbm, o_hbm):
        def load_idx(ch, b):
            pltpu.sync_copy(i_hbm.at[pl.ds(base + ch * CHUNK, CHUNK)], idx_b[b])

        def gstart(b):
            pltpu.make_async_copy(tab_sh.at[idx_b[b]], row_b[b], sem_g[b]).start()

        def gwait(b):
            pltpu.make_async_copy(tab_sh.at[idx_b[b]], row_b[b], sem_g[b]).wait()

        def wstart(ch, b):
            pltpu.make_async_copy(row_b[b], o_hbm.at[pl.ds(base + ch * CHUNK, CHUNK)],
                                  sem_w[b]).start()

        def wwait(b):
            pltpu.make_async_copy(row_b[b], o_hbm.at[pl.ds(base, CHUNK)],
                                  sem_w[b]).wait()

        load_idx(0, 0)
        gstart(0)

        n_loop = NCHK2 - 1 if NCHK2 % 2 == 1 else NCHK2 - 2

        @pl.loop(0, n_loop, step=2)
        def _(ch):
            load_idx(ch + 1, 1)
            gwait(0)

            @pl.when(ch > 0)
            def _():
                wwait(1)

            gstart(1)
            wstart(ch, 0)
            load_idx(ch + 2, 0)
            gwait(1)
            wwait(0)
            gstart(0)
            wstart(ch + 1, 1)

        if NCHK2 % 2 == 1:
            gwait(0)
            wwait(1)
            wstart(NCHK2 - 1, 0)
            wwait(0)
        else:
            load_idx(NCHK2 - 1, 1)
            gwait(0)
            wwait(1)
            gstart(1)
            wstart(NCHK2 - 2, 0)
            gwait(1)
            wwait(0)
            wstart(NCHK2 - 1, 1)
            wwait(1)

    @pl.when(cid == 0)
    def _():
        ring(src_hbm, gs_hbm)

    @pl.when(cid == 1)
    def _():
        ring(trm_hbm, gt_hbm)


_sc_gather = pl.kernel(
    _gather_body,
    out_type=(jax.ShapeDtypeStruct((EK, D), jnp.int32),
              jax.ShapeDtypeStruct((EK, D), jnp.int32)),
    mesh=_SC_MESH,
    scratch_types=[
        pltpu.VMEM_SHARED((V_PAD, D), jnp.int32),
        pltpu.VMEM((CHUNK,), jnp.int32),
        pltpu.VMEM((CHUNK,), jnp.int32),
        pltpu.VMEM((CHUNK, D), jnp.int32),
        pltpu.VMEM((CHUNK, D), jnp.int32),
        pltpu.SemaphoreType.DMA,
        pltpu.SemaphoreType.DMA,
        pltpu.SemaphoreType.DMA,
        pltpu.SemaphoreType.DMA,
    ],
)


def _unpack_pair(xi):
    lo = lax.bitcast_convert_type((xi & 0xFFFF).astype(jnp.uint16), jnp.bfloat16)
    hi = lax.bitcast_convert_type((xi >> 16).astype(jnp.uint16), jnp.bfloat16)
    return jnp.concatenate([lo, hi], axis=1)


def _mlp_body(gs_ref, gt_ref, e_ref, w1e_ref, w2n_ref, w2e_ref,
              b2n_ref, b2e_ref, y_ref, oute_ref):
    ps = _unpack_pair(gs_ref[...])
    qt = _unpack_pair(gt_ref[...])
    eb = e_ref[...].astype(jnp.bfloat16)
    ew = jnp.dot(eb, w1e_ref[...], preferred_element_type=jnp.float32)
    h = ew + ps.astype(jnp.float32) + qt.astype(jnp.float32)
    h = jnp.maximum(h, 0.0).astype(jnp.bfloat16)
    y_ref[...] = jnp.dot(h[:, :D], w2n_ref[...],
                         preferred_element_type=jnp.float32) + b2n_ref[...]
    ue = jnp.dot(h[:, D:], w2e_ref[...],
                 preferred_element_type=jnp.float32) + b2e_ref[...]
    oute_ref[...] = jnp.maximum(ue, 0.0)


def _run_mlp_chunk(k, gs, gt, e, w1e, w2n, w2e, b2n, b2e,
                   y_in=None, oute_in=None, block_e=2000):
    grid = (EK // block_e,)
    full = lambda i: (0, 0)
    blk0 = k * (EK // block_e)
    in_specs = [
        pl.BlockSpec((block_e, D), lambda i: (i, 0)),
        pl.BlockSpec((block_e, D), lambda i: (i, 0)),
        pl.BlockSpec((block_e, ED), lambda i: (i + blk0, 0)),
        pl.BlockSpec((ED, 2 * D), full),
        pl.BlockSpec((D, D), full),
        pl.BlockSpec((D, D), full),
        pl.BlockSpec((1, D), full),
        pl.BlockSpec((1, D), full),
    ]
    args = [gs, gt, e, w1e, w2n, w2e, b2n, b2e]
    kwargs = {}
    if y_in is not None:
        in_specs += [pl.BlockSpec(memory_space=pl.ANY),
                     pl.BlockSpec(memory_space=pl.ANY)]
        args += [y_in, oute_in]
        kwargs["input_output_aliases"] = {8: 0, 9: 1}

    def body(*refs):
        _mlp_body(*refs[:8], *refs[-2:])

    return pl.pallas_call(
        body,
        grid=grid,
        in_specs=in_specs,
        out_specs=[
            pl.BlockSpec((block_e, D), lambda i: (i + blk0, 0)),
            pl.BlockSpec((block_e, D), lambda i: (i + blk0, 0)),
        ],
        out_shape=[
            jax.ShapeDtypeStruct((E, D), jnp.float32),
            jax.ShapeDtypeStruct((E, D), jnp.float32),
        ],
        **kwargs,
    )(*args)


_ONES16 = None


def _scatter_body(y_hbm, trm_hbm, zrows_hbm, zdeg_hbm, out_hbm, deg_hbm,
                  acc_sh, idx_v0, idx_v1, rows_v0, rows_v1, deg_v,
                  sem_l0, sem_l1, sem_a0, sem_a1):
    cid = lax.axis_index("c")
    sid = lax.axis_index("s")
    wid = sid * NC + cid
    base = wid * EPW
    idx_b = (idx_v0, idx_v1)
    row_b = (rows_v0, rows_v1)
    sem_l = (sem_l0, sem_l1)
    sem_a = (sem_a0, sem_a1)
    ones = jnp.ones((16,), jnp.float32)

    def lstart(ch, b):
        off = base + ch * CHUNK
        pltpu.make_async_copy(trm_hbm.at[pl.ds(off, CHUNK)], idx_b[b], sem_l[b]).start()
        pltpu.make_async_copy(y_hbm.at[pl.ds(off, CHUNK)], row_b[b], sem_l[b]).start()

    def lwait(b):
        pltpu.make_async_copy(trm_hbm.at[pl.ds(base, CHUNK)], idx_b[b], sem_l[b]).wait()
        pltpu.make_async_copy(y_hbm.at[pl.ds(base, CHUNK)], row_b[b], sem_l[b]).wait()

    def astart(b):
        pltpu.async_copy(row_b[b], acc_sh.at[idx_b[b]], sem_a[b], add=True)

    def await_(b):
        pltpu.make_async_copy(row_b[b], acc_sh.at[idx_b[b]], sem_a[b]).wait()

    def deg_update(b):
        for j in range(CHUNK // 16):
            iv = idx_b[b][pl.ds(j * 16, 16)]
            plsc.addupdate_scatter(deg_v, [iv], ones)

    pltpu.sync_copy(zrows_hbm, acc_sh.at[pl.ds(sid * VPS, VPS)])
    pltpu.sync_copy(zdeg_hbm, deg_v)
    plsc.subcore_barrier()

    lstart(0, 0)

    @pl.loop(0, NCH - 1, step=2)
    def _(ch):
        lwait(0)

        @pl.when(ch > 0)
        def _():
            await_(1)

        lstart(ch + 1, 1)
        astart(0)
        deg_update(0)
        lwait(1)
        await_(0)
        lstart(ch + 2, 0)
        astart(1)
        deg_update(1)

    lwait(0)
    await_(1)
    astart(0)
    deg_update(0)
    await_(0)

    plsc.subcore_barrier()
    pltpu.sync_copy(acc_sh.at[pl.ds(sid * VPS, VPS)],
                    out_hbm.at[cid, pl.ds(sid * VPS, VPS)])
    pltpu.sync_copy(deg_v, deg_hbm.at[pl.ds(wid * V_PAD, V_PAD)])


_sc_scatter = pl.kernel(
    _scatter_body,
    out_type=(jax.ShapeDtypeStruct((NC, V_PAD, D), jnp.float32),
              jax.ShapeDtypeStruct((NW * V_PAD,), jnp.float32)),
    mesh=_SC_MESH,
    scratch_types=[
        pltpu.VMEM_SHARED((V_PAD, D), jnp.float32),
        pltpu.VMEM((CHUNK,), jnp.int32),
        pltpu.VMEM((CHUNK,), jnp.int32),
        pltpu.VMEM((CHUNK, D), jnp.float32),
        pltpu.VMEM((CHUNK, D), jnp.float32),
        pltpu.VMEM((V_PAD,), jnp.float32),
        pltpu.SemaphoreType.DMA,
        pltpu.SemaphoreType.DMA,
        pltpu.SemaphoreType.DMA,
        pltpu.SemaphoreType.DMA,
    ],
    compiler_params=_SC_PARAMS,
)


def _final_body(p_ref, degp_ref, out_ref):
    s = p_ref[0] + p_ref[1]
    ones_col = jnp.ones((NW, 1), jnp.float32)
    dcol = lax.dot_general(degp_ref[...], ones_col, (((0,), (0,)), ((), ())),
                           preferred_element_type=jnp.float32)
    deg = jnp.maximum(dcol, 1.0)
    out_ref[...] = jnp.maximum(s / deg, 0.0)


def _run_final(partials, degp, block_v=1280):
    return pl.pallas_call(
        _final_body,
        grid=(V_PAD // block_v,),
        in_specs=[
            pl.BlockSpec((NC, block_v, D), lambda i: (0, i, 0)),
            pl.BlockSpec((NW, block_v), lambda i: (0, i)),
        ],
        out_specs=pl.BlockSpec((block_v, D), lambda i: (i, 0)),
        out_shape=jax.ShapeDtypeStruct((V_PAD, D), jnp.float32),
    )(partials, degp)


def kernel(v, e, W1n, b1n, W2n, b2n, W1e, b1e, W2e, b2e, edge_list, num_nodes):
    v2 = v.reshape(V, D)
    e2 = e.reshape(E, ED)
    sour = edge_list[0].astype(jnp.int32)
    term = edge_list[1].astype(jnp.int32)

    ws = jnp.concatenate([W1n[:D], W1e[:D]], axis=1).astype(jnp.bfloat16)
    wt = jnp.concatenate([W1n[D:2 * D], W1e[D:2 * D]], axis=1).astype(jnp.bfloat16)
    w1e = jnp.concatenate([W1n[2 * D:], W1e[2 * D:]], axis=1).astype(jnp.bfloat16)
    b1 = jnp.concatenate([b1n, b1e]).reshape(1, 2 * D)
    b2n2 = b2n.reshape(1, D)
    b2e2 = b2e.reshape(1, D)
    W2nb = W2n.astype(jnp.bfloat16)
    W2eb = W2e.astype(jnp.bfloat16)

    vpad = jnp.pad(v2, ((0, V_PAD - V), (0, 0)))
    pp, qp = _run_pre(vpad, ws, wt, b1)
    y = out_e = None
    for k in range(K):
        sour_k = lax.slice(sour, (k * EK,), ((k + 1) * EK,))
        term_k = lax.slice(term, (k * EK,), ((k + 1) * EK,))
        gs_k, gt_k = _sc_gather(pp, qp, sour_k, term_k)
        y, out_e = _run_mlp_chunk(k, gs_k, gt_k, e2, w1e,
                                  W2nb, W2eb, b2n2, b2e2, y, out_e)
    zrows = jnp.zeros((VPS, D), jnp.float32)
    zdeg = jnp.zeros((V_PAD,), jnp.float32)
    partials, degp = _sc_scatter(y, term, zrows, zdeg)
    out_v = _run_final(partials, degp.reshape(NW, V_PAD))[:V]
    return out_v.reshape(1, V, D), out_e.reshape(1, E, D)


# speedup vs baseline: 4.4270x; 1.0756x over previous
import dataclasses

import jax
import jax.numpy as jnp
from jax import lax
from jax.experimental import pallas as pl
from jax.experimental.pallas import tpu as pltpu
from jax.experimental.pallas import tpu_sc as plsc

V = 10000
E = 320000
D = 128
ED = 16
NC = 2
NS = 16
NW = NC * NS
K = 5
EK = E // K
EPWK = EK // NW
EPW = E // NW
CHUNK = 80
NCHK = EPWK // CHUNK
NCH = EPW // CHUNK
V_PAD = 10240
VPS = V_PAD // NS

_SC_MESH = plsc.VectorSubcoreMesh(core_axis_name="c", subcore_axis_name="s")

_SC_PARAMS = pltpu.CompilerParams()
if "needs_layout_passes" in pltpu.CompilerParams.__dataclass_fields__:
    _SC_PARAMS = dataclasses.replace(_SC_PARAMS, needs_layout_passes=False)


def _pack_pair(x):
    xb = x.astype(jnp.bfloat16)
    lo = lax.bitcast_convert_type(xb[:, :D], jnp.uint16).astype(jnp.uint32)
    hi = lax.bitcast_convert_type(xb[:, D:], jnp.uint16).astype(jnp.uint32)
    return lax.bitcast_convert_type(lo | (hi << 16), jnp.int32)


def _pre_body(v_ref, ws_ref, wt_ref, b1_ref, pp_ref, qp_ref):
    xb = v_ref[...].astype(jnp.bfloat16)
    p = jnp.dot(xb, ws_ref[...], preferred_element_type=jnp.float32) + b1_ref[...]
    q = jnp.dot(xb, wt_ref[...], preferred_element_type=jnp.float32)
    pp_ref[...] = _pack_pair(p)
    qp_ref[...] = _pack_pair(q)


def _run_pre(vpad, ws, wt, b1, block_v=2048):
    full = lambda i: (0, 0)
    return pl.pallas_call(
        _pre_body,
        grid=(V_PAD // block_v,),
        in_specs=[
            pl.BlockSpec((block_v, D), lambda i: (i, 0)),
            pl.BlockSpec((D, 2 * D), full),
            pl.BlockSpec((D, 2 * D), full),
            pl.BlockSpec((1, 2 * D), full),
        ],
        out_specs=[
            pl.BlockSpec((block_v, D), lambda i: (i, 0)),
            pl.BlockSpec((block_v, D), lambda i: (i, 0)),
        ],
        out_shape=[
            jax.ShapeDtypeStruct((V_PAD, D), jnp.int32),
            jax.ShapeDtypeStruct((V_PAD, D), jnp.int32),
        ],
    )(vpad, ws, wt, b1)


EPWK2 = EK // NS
NCHK2 = EPWK2 // CHUNK


def _gather_body(p_hbm, q_hbm, src_hbm, trm_hbm, gs_hbm, gt_hbm,
                 tab_sh, idx0, idx1, rows0, rows1,
                 sem_g0, sem_g1, sem_w0, sem_w1):
    cid = lax.axis_index("c")
    sid = lax.axis_index("s")
    base = sid * EPWK2
    idx_b = (idx0, idx1)
    row_b = (rows0, rows1)
    sem_g = (sem_g0, sem_g1)
    sem_w = (sem_w0, sem_w1)

    @pl.when(cid == 0)
    def _():
        pltpu.sync_copy(p_hbm.at[pl.ds(sid * VPS, VPS)],
                        tab_sh.at[pl.ds(sid * VPS, VPS)])

    @pl.when(cid == 1)
    def _():
        pltpu.sync_copy(q_hbm.at[pl.ds(sid * VPS, VPS)],
                        tab_sh.at[pl.ds(sid * VPS, VPS)])

    plsc.subcore_barrier()

    def ring(i_hbm, o_hbm):
        def load_idx(ch, b):
            pltpu.sync_copy(i_hbm.at[pl.ds(base + ch * CHUNK, CHUNK)], idx_b[b])

        def gstart(b):
            pltpu.make_async_copy(tab_sh.at[idx_b[b]], row_b[b], sem_g[b]).start()

        def gwait(b):
            pltpu.make_async_copy(tab_sh.at[idx_b[b]], row_b[b], sem_g[b]).wait()

        def wstart(ch, b):
            pltpu.make_async_copy(row_b[b], o_hbm.at[pl.ds(base + ch * CHUNK, CHUNK)],
                                  sem_w[b]).start()

        def wwait(b):
            pltpu.make_async_copy(row_b[b], o_hbm.at[pl.ds(base, CHUNK)],
                                  sem_w[b]).wait()

        load_idx(0, 0)
        gstart(0)

        n_loop = NCHK2 - 1 if NCHK2 % 2 == 1 else NCHK2 - 2

        @pl.loop(0, n_loop, step=2)
        def _(ch):
            load_idx(ch + 1, 1)
            gwait(0)

            @pl.when(ch > 0)
            def _():
                wwait(1)

            gstart(1)
            wstart(ch, 0)
            load_idx(ch + 2, 0)
            gwait(1)
            wwait(0)
            gstart(0)
            wstart(ch + 1, 1)

        if NCHK2 % 2 == 1:
            gwait(0)
            wwait(1)
            wstart(NCHK2 - 1, 0)
            wwait(0)
        else:
            load_idx(NCHK2 - 1, 1)
            gwait(0)
            wwait(1)
            gstart(1)
            wstart(NCHK2 - 2, 0)
            gwait(1)
            wwait(0)
            wstart(NCHK2 - 1, 1)
            wwait(1)

    @pl.when(cid == 0)
    def _():
        ring(src_hbm, gs_hbm)

    @pl.when(cid == 1)
    def _():
        ring(trm_hbm, gt_hbm)


_sc_gather = pl.kernel(
    _gather_body,
    out_type=(jax.ShapeDtypeStruct((EK, D), jnp.int32),
              jax.ShapeDtypeStruct((EK, D), jnp.int32)),
    mesh=_SC_MESH,
    scratch_types=[
        pltpu.VMEM_SHARED((V_PAD, D), jnp.int32),
        pltpu.VMEM((CHUNK,), jnp.int32),
        pltpu.VMEM((CHUNK,), jnp.int32),
        pltpu.VMEM((CHUNK, D), jnp.int32),
        pltpu.VMEM((CHUNK, D), jnp.int32),
        pltpu.SemaphoreType.DMA,
        pltpu.SemaphoreType.DMA,
        pltpu.SemaphoreType.DMA,
        pltpu.SemaphoreType.DMA,
    ],
)


def _unpack_pair(xi):
    lo = lax.bitcast_convert_type((xi & 0xFFFF).astype(jnp.uint16), jnp.bfloat16)
    hi = lax.bitcast_convert_type((xi >> 16).astype(jnp.uint16), jnp.bfloat16)
    return jnp.concatenate([lo, hi], axis=1)


def _mlp_body(gs_ref, gt_ref, e_ref, w1e_ref, w2n_ref, w2e_ref,
              b2n_ref, b2e_ref, y_ref, oute_ref):
    ps = _unpack_pair(gs_ref[...])
    qt = _unpack_pair(gt_ref[...])
    eb = e_ref[...].astype(jnp.bfloat16)
    ew = jnp.dot(eb, w1e_ref[...], preferred_element_type=jnp.float32)
    h = ew + ps.astype(jnp.float32) + qt.astype(jnp.float32)
    h = jnp.maximum(h, 0.0).astype(jnp.bfloat16)
    y_ref[...] = jnp.dot(h[:, :D], w2n_ref[...],
                         preferred_element_type=jnp.float32) + b2n_ref[...]
    ue = jnp.dot(h[:, D:], w2e_ref[...],
                 preferred_element_type=jnp.float32) + b2e_ref[...]
    oute_ref[...] = jnp.maximum(ue, 0.0)


def _run_mlp_chunk(k, gs, gt, e, w1e, w2n, w2e, b2n, b2e,
                   oute_in=None, block_e=2000):
    grid = (EK // block_e,)
    full = lambda i: (0, 0)
    blk0 = k * (EK // block_e)
    in_specs = [
        pl.BlockSpec((block_e, D), lambda i: (i, 0)),
        pl.BlockSpec((block_e, D), lambda i: (i, 0)),
        pl.BlockSpec((block_e, ED), lambda i: (i + blk0, 0)),
        pl.BlockSpec((ED, 2 * D), full),
        pl.BlockSpec((D, D), full),
        pl.BlockSpec((D, D), full),
        pl.BlockSpec((1, D), full),
        pl.BlockSpec((1, D), full),
    ]
    args = [gs, gt, e, w1e, w2n, w2e, b2n, b2e]
    kwargs = {}
    if oute_in is not None:
        in_specs += [pl.BlockSpec(memory_space=pl.ANY)]
        args += [oute_in]
        kwargs["input_output_aliases"] = {8: 1}

    def body(*refs):
        _mlp_body(*refs[:8], *refs[-2:])

    return pl.pallas_call(
        body,
        grid=grid,
        in_specs=in_specs,
        out_specs=[
            pl.BlockSpec((block_e, D), lambda i: (i, 0)),
            pl.BlockSpec((block_e, D), lambda i: (i + blk0, 0)),
        ],
        out_shape=[
            jax.ShapeDtypeStruct((EK, D), jnp.float32),
            jax.ShapeDtypeStruct((E, D), jnp.float32),
        ],
        **kwargs,
    )(*args)


EPA = EK // NW
NCHA = EPA // CHUNK


def _make_scatter(edge0, n_arr):
    def body(*refs):
        ys = refs[:n_arr]
        (trm_hbm, zrows_hbm, zdeg_hbm, out_hbm, deg_hbm,
         acc_sh, idx_v0, idx_v1, rows_v0, rows_v1, deg_v,
         sem_l0, sem_l1, sem_a0, sem_a1) = refs[n_arr:]
        cid = lax.axis_index("c")
        sid = lax.axis_index("s")
        wid = sid * NC + cid
        wbase = wid * EPA
        idx_b = (idx_v0, idx_v1)
        row_b = (rows_v0, rows_v1)
        sem_l = (sem_l0, sem_l1)
        sem_a = (sem_a0, sem_a1)
        ones = jnp.ones((16,), jnp.float32)

        def lwait(b):
            pltpu.make_async_copy(trm_hbm.at[pl.ds(wbase, CHUNK)], idx_b[b], sem_l[b]).wait()
            pltpu.make_async_copy(ys[0].at[pl.ds(wbase, CHUNK)], row_b[b], sem_l[b]).wait()

        def astart(b):
            pltpu.async_copy(row_b[b], acc_sh.at[idx_b[b]], sem_a[b], add=True)

        def await_(b):
            pltpu.make_async_copy(row_b[b], acc_sh.at[idx_b[b]], sem_a[b]).wait()

        def deg_update(b):
            for j in range(CHUNK // 16):
                iv = idx_b[b][pl.ds(j * 16, 16)]
                plsc.addupdate_scatter(deg_v, [iv], ones)

        pltpu.sync_copy(zrows_hbm, acc_sh.at[pl.ds(sid * VPS, VPS)])
        pltpu.sync_copy(zdeg_hbm, deg_v)
        plsc.subcore_barrier()

        for a in range(n_arr):
            y_hbm = ys[a]

            def lstart(ch, b, a=a, y_hbm=y_hbm):
                ioff = edge0 + a * EK + wbase + ch * CHUNK
                roff = wbase + ch * CHUNK
                pltpu.make_async_copy(trm_hbm.at[pl.ds(ioff, CHUNK)],
                                      idx_b[b], sem_l[b]).start()
                pltpu.make_async_copy(y_hbm.at[pl.ds(roff, CHUNK)],
                                      row_b[b], sem_l[b]).start()

            lstart(0, 0)

            @pl.loop(0, NCHA - 1, step=2)
            def _(ch, lstart=lstart):
                lwait(0)

                @pl.when(ch > 0)
                def _():
                    await_(1)

                lstart(ch + 1, 1)
                astart(0)
                deg_update(0)
                lwait(1)
                await_(0)
                lstart(ch + 2, 0)
                astart(1)
                deg_update(1)

            lwait(0)
            await_(1)
            astart(0)
            deg_update(0)
            await_(0)

        plsc.subcore_barrier()
        pltpu.sync_copy(acc_sh.at[pl.ds(sid * VPS, VPS)],
                        out_hbm.at[cid, pl.ds(sid * VPS, VPS)])
        pltpu.sync_copy(deg_v, deg_hbm.at[pl.ds(wid * V_PAD, V_PAD)])

    return pl.kernel(
        body,
        out_type=(jax.ShapeDtypeStruct((NC, V_PAD, D), jnp.float32),
                  jax.ShapeDtypeStruct((NW * V_PAD,), jnp.float32)),
        mesh=_SC_MESH,
        scratch_types=[
            pltpu.VMEM_SHARED((V_PAD, D), jnp.float32),
            pltpu.VMEM((CHUNK,), jnp.int32),
            pltpu.VMEM((CHUNK,), jnp.int32),
            pltpu.VMEM((CHUNK, D), jnp.float32),
            pltpu.VMEM((CHUNK, D), jnp.float32),
            pltpu.VMEM((V_PAD,), jnp.float32),
            pltpu.SemaphoreType.DMA,
            pltpu.SemaphoreType.DMA,
            pltpu.SemaphoreType.DMA,
            pltpu.SemaphoreType.DMA,
        ],
        compiler_params=_SC_PARAMS,
    )


_KA = 3
_sc_scatter_a = _make_scatter(0, _KA)
_sc_scatter_b = _make_scatter(_KA * EK, K - _KA)


def _final_body(pa_ref, pb_ref, dega_ref, degb_ref, out_ref):
    s = pa_ref[0] + pa_ref[1] + pb_ref[0] + pb_ref[1]
    ones_col = jnp.ones((NW, 1), jnp.float32)
    dcol = (lax.dot_general(dega_ref[...], ones_col, (((0,), (0,)), ((), ())),
                            preferred_element_type=jnp.float32)
            + lax.dot_general(degb_ref[...], ones_col, (((0,), (0,)), ((), ())),
                              preferred_element_type=jnp.float32))
    deg = jnp.maximum(dcol, 1.0)
    out_ref[...] = jnp.maximum(s / deg, 0.0)


def _run_final(pa, pb, dega, degb, block_v=1280):
    return pl.pallas_call(
        _final_body,
        grid=(V_PAD // block_v,),
        in_specs=[
            pl.BlockSpec((NC, block_v, D), lambda i: (0, i, 0)),
            pl.BlockSpec((NC, block_v, D), lambda i: (0, i, 0)),
            pl.BlockSpec((NW, block_v), lambda i: (0, i)),
            pl.BlockSpec((NW, block_v), lambda i: (0, i)),
        ],
        out_specs=pl.BlockSpec((block_v, D), lambda i: (i, 0)),
        out_shape=jax.ShapeDtypeStruct((V_PAD, D), jnp.float32),
    )(pa, pb, dega, degb)


def kernel(v, e, W1n, b1n, W2n, b2n, W1e, b1e, W2e, b2e, edge_list, num_nodes):
    v2 = v.reshape(V, D)
    e2 = e.reshape(E, ED)
    sour = edge_list[0].astype(jnp.int32)
    term = edge_list[1].astype(jnp.int32)

    ws = jnp.concatenate([W1n[:D], W1e[:D]], axis=1).astype(jnp.bfloat16)
    wt = jnp.concatenate([W1n[D:2 * D], W1e[D:2 * D]], axis=1).astype(jnp.bfloat16)
    w1e = jnp.concatenate([W1n[2 * D:], W1e[2 * D:]], axis=1).astype(jnp.bfloat16)
    b1 = jnp.concatenate([b1n, b1e]).reshape(1, 2 * D)
    b2n2 = b2n.reshape(1, D)
    b2e2 = b2e.reshape(1, D)
    W2nb = W2n.astype(jnp.bfloat16)
    W2eb = W2e.astype(jnp.bfloat16)

    vpad = jnp.pad(v2, ((0, V_PAD - V), (0, 0)))
    pp, qp = _run_pre(vpad, ws, wt, b1)
    ys = []
    out_e = None
    for k in range(K):
        sour_k = lax.slice(sour, (k * EK,), ((k + 1) * EK,))
        term_k = lax.slice(term, (k * EK,), ((k + 1) * EK,))
        gs_k, gt_k = _sc_gather(pp, qp, sour_k, term_k)
        y_k, out_e = _run_mlp_chunk(k, gs_k, gt_k, e2, w1e,
                                    W2nb, W2eb, b2n2, b2e2, out_e)
        ys.append(y_k)
    zrows = jnp.zeros((VPS, D), jnp.float32)
    zdeg = jnp.zeros((V_PAD,), jnp.float32)
    pa, dega = _sc_scatter_a(*ys[:_KA], term, zrows, zdeg)
    pb, degb = _sc_scatter_b(*ys[_KA:], term, zrows, zdeg)
    out_v = _run_final(pa, pb, dega.reshape(NW, V_PAD),
                       degb.reshape(NW, V_PAD))[:V]
    return out_v.reshape(1, V, D), out_e.reshape(1, E, D)
